# baseline (speedup 1.0000x reference)
import jax
import jax.numpy as jnp
from jax import lax
from jax.experimental import pallas as pl
from jax.experimental.pallas import tpu as pltpu
from jax.experimental.pallas import tpu_sc as plsc

_S = 512
_B = 2048
_N2 = 102400
_NW = 32
_CHUNK = _N2 // _NW
_T = 64
_H = 4


def _tc_pass1(x_ref, seg_ref, wt_ref, b_ref, scores_ref, m_ref, denom_ref):
    i = pl.program_id(0)

    @pl.when(i == 0)
    def _init():
        m_ref[...] = jnp.full_like(m_ref, -jnp.inf)
        denom_ref[...] = jnp.zeros_like(denom_ref)

    x = x_ref[...]
    s = jnp.dot(x, wt_ref[...], preferred_element_type=jnp.float32) + b_ref[...]
    scores_ref[...] = s

    m_old = m_ref[0:1, :]
    m_new = jnp.maximum(m_old, jnp.max(s, axis=0, keepdims=True))
    scale = jnp.where(m_new == m_old, 1.0, jnp.exp(m_old - m_new))
    e = jnp.exp(s - m_new)

    seg = seg_ref[0, 0, :]
    ot = (jax.lax.broadcasted_iota(jnp.int32, (_S, s.shape[0]), 0)
          == seg[None, :]).astype(jnp.float32)
    dblk = jnp.dot(ot, e, preferred_element_type=jnp.float32)
    denom_ref[...] = denom_ref[...] * scale + dblk
    m_ref[...] = jnp.broadcast_to(m_new, m_ref.shape)


_PR = _S + 2
_PF = _PR * 128
_SLICE = (_S * 128) // 16


def _sc_body(scores_hbm, seg_hbm, m_hbm, den_hbm, x_hbm, zeros_hbm,
             attn_hbm, pooled_hbm,
             sc_v, seg_v, m_v, den_v, r_v, attn_b, c_v, x_b, pool_f):
    cid = lax.axis_index("c")
    sid = lax.axis_index("s")
    wid = cid * 16 + sid
    base = wid * _CHUNK

    pltpu.sync_copy(scores_hbm.at[pl.ds(base * _H, _CHUNK * _H)], sc_v)
    pltpu.sync_copy(seg_hbm.at[pl.ds(base, _CHUNK)], seg_v)
    pltpu.sync_copy(m_hbm, m_v)
    pltpu.sync_copy(den_hbm, den_v)
    pltpu.sync_copy(zeros_hbm, pool_f)

    iota = lax.iota(jnp.int32, 16)
    iota4 = iota * 4

    def _rloop(k, carry):
        dv = den_v[pl.ds(k * 16, 16)]
        r_v[pl.ds(k * 16, 16)] = 1.0 / jnp.maximum(dv, 1e-16)
        return carry
    lax.fori_loop(0, (_S * _H) // 16, _rloop, 0)
    r_v[pl.ds(_S * _H, 16)] = jnp.zeros((16,), jnp.float32)

    m_splat = [plsc.load_gather(m_v, [jnp.full((16,), h + _H, jnp.int32)])
               for h in range(_H)]

    def _aloop(k, carry):
        segs = seg_v[pl.ds(k * 16, 16)]
        c16 = jnp.zeros((16,), jnp.float32)
        for h in range(_H):
            s_h = plsc.load_gather(sc_v, [k * 64 + iota4 + h])
            e = jnp.exp(s_h - m_splat[h])
            rg = plsc.load_gather(r_v, [segs * _H + h])
            a = e * rg
            attn_b[pl.ds(h * _CHUNK + k * 16, 16)] = a
            c16 = c16 + a
        c_v[pl.ds(k * 16, 16)] = 0.25 * c16
        return carry
    lax.fori_loop(0, _CHUNK // 16, _aloop, 0)

    for h in range(_H):
        pltpu.sync_copy(attn_b.at[pl.ds(h * _CHUNK, _CHUNK)],
                        attn_hbm.at[h, pl.ds(base, _CHUNK)])

    def _ploop(t, carry):
        pltpu.sync_copy(x_hbm.at[pl.ds(base + t * _T, _T)], x_b)

        def _jloop(j, c2):
            nl = t * _T + j
            cj = plsc.load_gather(c_v, [jnp.full((16,), nl, jnp.int32)])
            sb = plsc.load_gather(seg_v, [jnp.full((16,), nl, jnp.int32)])
            rowbase = sb * 128 + iota
            for q in range(8):
                v = x_b[j, pl.ds(q * 16, 16)] * cj
                plsc.addupdate_scatter(pool_f, [rowbase + q * 16], v)
            return c2
        lax.fori_loop(0, _T, _jloop, 0)
        return carry
    lax.fori_loop(0, _CHUNK // _T, _ploop, 0)

    pltpu.sync_copy(pool_f.at[pl.ds(0, _S * 128)], pooled_hbm.at[wid])


def _tc_combine(p_ref, out_ref):
    out_ref[...] = jnp.sum(p_ref[...], axis=0)


def kernel(x, batch_indices, W, b, temperature):
    n, d = x.shape
    h = W.shape[0]

    x_p = jnp.pad(x, ((0, _N2 - n), (0, 0)))
    seg_p = jnp.pad(batch_indices.astype(jnp.int32), (0, _N2 - n),
                    constant_values=_S)

    wt = (W / temperature).T.astype(jnp.float32)
    b2 = (b / temperature).reshape(1, h).astype(jnp.float32)
    seg3 = seg_p.reshape(_N2 // _B, 1, _B)

    params = pltpu.CompilerParams(dimension_semantics=("arbitrary",))
    nblk = _N2 // _B

    scores, m, denom = pl.pallas_call(
        _tc_pass1,
        grid=(nblk,),
        in_specs=[
            pl.BlockSpec((_B, d), lambda i: (i, 0)),
            pl.BlockSpec((1, 1, _B), lambda i: (i, 0, 0)),
            pl.BlockSpec((d, h), lambda i: (0, 0)),
            pl.BlockSpec((1, h), lambda i: (0, 0)),
        ],
        out_specs=[
            pl.BlockSpec((_B, h), lambda i: (i, 0)),
            pl.BlockSpec((8, h), lambda i: (0, 0)),
            pl.BlockSpec((_S, h), lambda i: (0, 0)),
        ],
        out_shape=[
            jax.ShapeDtypeStruct((_N2, h), jnp.float32),
            jax.ShapeDtypeStruct((8, h), jnp.float32),
            jax.ShapeDtypeStruct((_S, h), jnp.float32),
        ],
        compiler_params=params,
    )(x_p, seg3, wt, b2)

    mesh = plsc.VectorSubcoreMesh(core_axis_name="c", subcore_axis_name="s",
                                  num_cores=2, num_subcores=16)
    sc_call = pl.kernel(
        _sc_body,
        out_type=(
            jax.ShapeDtypeStruct((_H, _N2), jnp.float32),
            jax.ShapeDtypeStruct((_NW, _S * 128), jnp.float32),
        ),
        mesh=mesh,
        compiler_params=pltpu.CompilerParams(needs_layout_passes=False),
        scratch_types=[
            pltpu.VMEM((_CHUNK * _H,), jnp.float32),
            pltpu.VMEM((_CHUNK,), jnp.int32),
            pltpu.VMEM((32,), jnp.float32),
            pltpu.VMEM((_S * _H,), jnp.float32),
            pltpu.VMEM((_S * _H + 128,), jnp.float32),
            pltpu.VMEM((_CHUNK * _H,), jnp.float32),
            pltpu.VMEM((_CHUNK,), jnp.float32),
            pltpu.VMEM((_T, d), jnp.float32),
            pltpu.VMEM((_PF,), jnp.float32),
        ],
    )
    attn_t, pooled_p = sc_call(
        scores.reshape(-1), seg_p, m.reshape(-1), denom.reshape(-1), x_p,
        jnp.zeros((_PF,), jnp.float32))

    pooled = pl.pallas_call(
        _tc_combine,
        out_shape=jax.ShapeDtypeStruct((_S, d), jnp.float32),
    )(pooled_p.reshape(_NW, _S, d))

    return (pooled, attn_t[:, :n])


# speedup vs baseline: 1.4031x; 1.4031x over previous
import jax
import jax.numpy as jnp
from jax import lax
from jax.experimental import pallas as pl
from jax.experimental.pallas import tpu as pltpu
from jax.experimental.pallas import tpu_sc as plsc

_S = 512
_B = 2048
_N2 = 102400
_NW = 32
_CHUNK = _N2 // _NW
_T = 64
_H = 4


def _tc_pass1(x_ref, seg_ref, wt_ref, b_ref, scores_ref, m_ref, denom_ref):
    i = pl.program_id(0)

    @pl.when(i == 0)
    def _init():
        m_ref[...] = jnp.full_like(m_ref, -jnp.inf)
        denom_ref[...] = jnp.zeros_like(denom_ref)

    x = x_ref[...]
    s = jnp.dot(x, wt_ref[...], preferred_element_type=jnp.float32) + b_ref[...]
    scores_ref[...] = s

    m_old = m_ref[0:1, :]
    m_new = jnp.maximum(m_old, jnp.max(s, axis=0, keepdims=True))
    scale = jnp.where(m_new == m_old, 1.0, jnp.exp(m_old - m_new))
    e = jnp.exp(s - m_new)

    seg = seg_ref[0, 0, :]
    ot = (jax.lax.broadcasted_iota(jnp.int32, (_S, s.shape[0]), 0)
          == seg[None, :]).astype(jnp.float32)
    dblk = jnp.dot(ot, e, preferred_element_type=jnp.float32)
    denom_ref[...] = denom_ref[...] * scale + dblk
    m_ref[...] = jnp.broadcast_to(m_new, m_ref.shape)


_PR = _S + 2
_PF = _PR * 128
_SLICE = (_S * 128) // 16


def _sc_body(scores_hbm, seg_hbm, m_hbm, den_hbm,
             attn_hbm, c_hbm,
             sc_v, seg_v, m_v, den_v, r_v, attn_b, c_v):
    cid = lax.axis_index("c")
    sid = lax.axis_index("s")
    wid = cid * 16 + sid
    base = wid * _CHUNK

    pltpu.sync_copy(scores_hbm.at[pl.ds(base * _H, _CHUNK * _H)], sc_v)
    pltpu.sync_copy(seg_hbm.at[pl.ds(base, _CHUNK)], seg_v)
    pltpu.sync_copy(m_hbm, m_v)
    pltpu.sync_copy(den_hbm, den_v)

    iota = lax.iota(jnp.int32, 16)
    iota4 = iota * 4

    def _rloop(k, carry):
        dv = den_v[pl.ds(k * 16, 16)]
        r_v[pl.ds(k * 16, 16)] = 1.0 / jnp.maximum(dv, 1e-16)
        return carry
    lax.fori_loop(0, (_S * _H) // 16, _rloop, 0)
    r_v[pl.ds(_S * _H, 16)] = jnp.zeros((16,), jnp.float32)

    m_splat = [plsc.load_gather(m_v, [jnp.full((16,), h + _H, jnp.int32)])
               for h in range(_H)]

    def _aloop(k, carry):
        segs = seg_v[pl.ds(k * 16, 16)]
        c16 = jnp.zeros((16,), jnp.float32)
        for h in range(_H):
            s_h = plsc.load_gather(sc_v, [k * 64 + iota4 + h])
            e = jnp.exp(s_h - m_splat[h])
            rg = plsc.load_gather(r_v, [segs * _H + h])
            a = e * rg
            attn_b[pl.ds(h * _CHUNK + k * 16, 16)] = a
            c16 = c16 + a
        c_v[pl.ds(k * 16, 16)] = 0.25 * c16
        return carry
    lax.fori_loop(0, _CHUNK // 16, _aloop, 0)

    for h in range(_H):
        pltpu.sync_copy(attn_b.at[pl.ds(h * _CHUNK, _CHUNK)],
                        attn_hbm.at[h, pl.ds(base, _CHUNK)])
    pltpu.sync_copy(c_v, c_hbm.at[pl.ds(base, _CHUNK)])


def _tc_pooled(x_ref, seg_ref, c_ref, pooled_ref):
    i = pl.program_id(0)

    @pl.when(i == 0)
    def _init():
        pooled_ref[...] = jnp.zeros_like(pooled_ref)

    seg = seg_ref[0, 0, :]
    ot = (jax.lax.broadcasted_iota(jnp.int32, (_S, seg.shape[0]), 0)
          == seg[None, :]).astype(jnp.float32)
    y = x_ref[...] * c_ref[...]
    pooled_ref[...] += jnp.dot(ot, y, preferred_element_type=jnp.float32)


def kernel(x, batch_indices, W, b, temperature):
    n, d = x.shape
    h = W.shape[0]

    x_p = jnp.pad(x, ((0, _N2 - n), (0, 0)))
    seg_p = jnp.pad(batch_indices.astype(jnp.int32), (0, _N2 - n),
                    constant_values=_S)

    wt = (W / temperature).T.astype(jnp.float32)
    b2 = (b / temperature).reshape(1, h).astype(jnp.float32)
    seg3 = seg_p.reshape(_N2 // _B, 1, _B)

    params = pltpu.CompilerParams(dimension_semantics=("arbitrary",))
    nblk = _N2 // _B

    scores, m, denom = pl.pallas_call(
        _tc_pass1,
        grid=(nblk,),
        in_specs=[
            pl.BlockSpec((_B, d), lambda i: (i, 0)),
            pl.BlockSpec((1, 1, _B), lambda i: (i, 0, 0)),
            pl.BlockSpec((d, h), lambda i: (0, 0)),
            pl.BlockSpec((1, h), lambda i: (0, 0)),
        ],
        out_specs=[
            pl.BlockSpec((_B, h), lambda i: (i, 0)),
            pl.BlockSpec((8, h), lambda i: (0, 0)),
            pl.BlockSpec((_S, h), lambda i: (0, 0)),
        ],
        out_shape=[
            jax.ShapeDtypeStruct((_N2, h), jnp.float32),
            jax.ShapeDtypeStruct((8, h), jnp.float32),
            jax.ShapeDtypeStruct((_S, h), jnp.float32),
        ],
        compiler_params=params,
    )(x_p, seg3, wt, b2)

    mesh = plsc.VectorSubcoreMesh(core_axis_name="c", subcore_axis_name="s",
                                  num_cores=2, num_subcores=16)
    sc_call = pl.kernel(
        _sc_body,
        out_type=(
            jax.ShapeDtypeStruct((_H, _N2), jnp.float32),
            jax.ShapeDtypeStruct((_N2,), jnp.float32),
        ),
        mesh=mesh,
        compiler_params=pltpu.CompilerParams(needs_layout_passes=False),
        scratch_types=[
            pltpu.VMEM((_CHUNK * _H,), jnp.float32),
            pltpu.VMEM((_CHUNK,), jnp.int32),
            pltpu.VMEM((32,), jnp.float32),
            pltpu.VMEM((_S * _H,), jnp.float32),
            pltpu.VMEM((_S * _H + 128,), jnp.float32),
            pltpu.VMEM((_CHUNK * _H,), jnp.float32),
            pltpu.VMEM((_CHUNK,), jnp.float32),
        ],
    )
    attn_t, c = sc_call(
        scores.reshape(-1), seg_p, m.reshape(-1), denom.reshape(-1))

    pooled = pl.pallas_call(
        _tc_pooled,
        grid=(nblk,),
        in_specs=[
            pl.BlockSpec((_B, d), lambda i: (i, 0)),
            pl.BlockSpec((1, 1, _B), lambda i: (i, 0, 0)),
            pl.BlockSpec((_B, 1), lambda i: (i, 0)),
        ],
        out_specs=pl.BlockSpec((_S, d), lambda i: (0, 0)),
        out_shape=jax.ShapeDtypeStruct((_S, d), jnp.float32),
        compiler_params=params,
    )(x_p, seg3, c.reshape(_N2, 1))

    return (pooled, attn_t[:, :n])
